# baseline (speedup 1.0000x reference)
import functools

import jax
import jax.numpy as jnp
import numpy as np
from jax.experimental import pallas as pl
from jax.experimental.pallas import tpu as pltpu
from jax.experimental.pallas import tpu_sc as plsc

B, T, D = 2, 2048, 1024
H, DH, K = 16, 64, 128
BH = B * H
RT = 8
TQ = 512
NEG = -1e30
L = 16
TCH = 256
SCALE = 1.0 / np.sqrt(DH)


def _proj_kernel(ltau_ref, x_ref, xbf_ref, wq_ref, wkbf_ref, wvbf_ref, sk_ref,
                 qsb_ref, kb_ref, vf_ref, lg_ref):
    xb = x_ref[0]
    xbf = xbf_ref[0]
    q2 = jax.lax.dot_general(xb, wq_ref[...], (((1,), (1,)), ((), ())),
                             preferred_element_type=jnp.float32)
    qsb_ref[0, 0] = (q2[:, :DH] * SCALE).astype(jnp.bfloat16)
    qsb_ref[0, 1] = (q2[:, DH:] * SCALE).astype(jnp.bfloat16)
    k2 = jax.lax.dot_general(xbf, wkbf_ref[...], (((1,), (1,)), ((), ())),
                             preferred_element_type=jnp.float32)
    kb_ref[0, 0] = k2[:, :DH].astype(jnp.bfloat16)
    kb_ref[0, 1] = k2[:, DH:].astype(jnp.bfloat16)
    v2 = jax.lax.dot_general(xbf, wvbf_ref[...], (((1,), (1,)), ((), ())),
                             preferred_element_type=jnp.float32)
    vf_ref[0, 0] = v2[:, :DH]
    vf_ref[0, 1] = v2[:, DH:]
    tau = jnp.exp(ltau_ref[0, 0])
    for hh in range(2):
        q = q2[:, hh * DH:(hh + 1) * DH]
        qn = q / jnp.maximum(jnp.sqrt(jnp.sum(q * q, axis=-1, keepdims=True)),
                             1e-12)
        sk = sk_ref[hh]
        sk = sk / jnp.maximum(jnp.sqrt(jnp.sum(sk * sk, axis=-1, keepdims=True)),
                              1e-12)
        lg_ref[hh] = jax.lax.dot_general(qn, sk, (((1,), (1,)), ((), ())),
                                         preferred_element_type=jnp.float32) * tau


def _bcast_lane(x, j):
    idx = jax.lax.broadcasted_iota(jnp.int32, (L,), 0) * 0 + j
    return jax.lax.gather(
        x, idx[:, None],
        jax.lax.GatherDimensionNumbers(offset_dims=(), collapsed_slice_dims=(0,),
                                       start_index_map=(0,)),
        (1,), mode=jax.lax.GatherScatterMode.PROMISE_IN_BOUNDS)


def _sc_read_body(lg_hbm, sv_hbm, out_hbm, lg_s, sv_s, ro_s, ib_s, eb_s, sem):
    c = jax.lax.axis_index("c")
    s = jax.lax.axis_index("s")
    wid = s * 2 + c
    head = jax.lax.rem(wid, H)
    pltpu.sync_copy(sv_hbm.at[head], sv_s)

    nvec = K // L

    def token_body(t, carry):
        lvec = [lg_s[t, pl.ds(i * L, L)] for i in range(nvec)]
        work = list(lvec)
        t8 = jnp.float32(0)
        for it in range(RT):
            m = work[0]
            for i in range(1, nvec):
                m = jnp.maximum(m, work[i])
            t8 = jnp.max(m)
            if it < RT - 1:
                mb = jnp.full((L,), t8, jnp.float32)
                work = [jnp.where(w >= mb, NEG, w) for w in work]
        t8b = jnp.full((L,), t8, jnp.float32)
        evec = [jnp.where(lv >= t8b, jnp.exp(lv - t8b), 0.0) for lv in lvec]
        den = evec[0]
        for i in range(1, nvec):
            den = den + evec[i]
        denb = jnp.full((L,), jnp.sum(den), jnp.float32)
        base = jnp.zeros((L,), jnp.int32)
        for i in range(nvec):
            msk = lvec[i] >= t8b
            mi = jnp.where(msk, 1, 0).astype(jnp.int32)
            rank = base + plsc.cumsum(mi) - 1
            iv = jax.lax.broadcasted_iota(jnp.int32, (L,), 0) + i * L
            plsc.store_scatter(ib_s, [rank], iv, mask=msk)
            plsc.store_scatter(eb_s, [rank], evec[i], mask=msk)
            base = base + plsc.all_reduce_population_count(msk)
        sel_i = ib_s[pl.ds(0, L)]
        sel_e = eb_s[pl.ds(0, L)]
        acc = [jnp.zeros((L,), jnp.float32) for _ in range(DH // L)]
        for j in range(RT):
            rowbase = _bcast_lane(sel_i, j) * DH
            ej = _bcast_lane(sel_e, j)
            for dc in range(DH // L):
                addr = rowbase + (dc * L + jax.lax.broadcasted_iota(jnp.int32, (L,), 0))
                acc[dc] = acc[dc] + ej * plsc.load_gather(sv_s, [addr])
        for dc in range(DH // L):
            ro_s[t, pl.ds(dc * L, L)] = acc[dc] / denb
        return carry

    for chunk in range(T // TCH):
        pltpu.sync_copy(lg_hbm.at[wid, pl.ds(chunk * TCH, TCH)], lg_s)
        jax.lax.fori_loop(0, TCH, token_body, jnp.int32(0))
        pltpu.sync_copy(ro_s, out_hbm.at[wid, pl.ds(chunk * TCH, TCH)])


_sc_read = functools.partial(
    pl.kernel,
    mesh=plsc.VectorSubcoreMesh(core_axis_name="c", subcore_axis_name="s"),
    compiler_params=pltpu.CompilerParams(needs_layout_passes=False),
    out_type=jax.ShapeDtypeStruct((BH, T, DH), jnp.float32),
    scratch_types=[
        pltpu.VMEM((TCH, K), jnp.float32),
        pltpu.VMEM((K * DH,), jnp.float32),
        pltpu.VMEM((TCH, DH), jnp.float32),
        pltpu.VMEM((160,), jnp.int32),
        pltpu.VMEM((160,), jnp.float32),
        pltpu.SemaphoreType.DMA,
    ],
)(_sc_read_body)


def _attn_kernel(qsb_ref, kb_ref, vf_ref, wfbf_ref, bf_ref, part_ref, vb_s):
    vb_s[:, :DH] = vf_ref[0, 0].astype(jnp.bfloat16)
    lane = jax.lax.broadcasted_iota(jnp.int32, (T, DH), 1)
    vb_s[:, DH:] = jnp.where(lane == 0, 1.0, 0.0).astype(jnp.bfloat16)

    qsb = qsb_ref[0, 0]
    gl = jax.lax.dot_general(qsb, wfbf_ref[...], (((1,), (1,)), ((), ())),
                             preferred_element_type=jnp.float32) / SCALE + bf_ref[...]
    gmx = jnp.max(gl, axis=-1, keepdims=True)
    ge = jnp.exp(gl - gmx)
    g = ge / jnp.sum(ge, axis=-1, keepdims=True)

    for qt in range(T // TQ):
        qtile = qsb_ref[0, 0, pl.ds(qt * TQ, TQ), :]

        def body(kt, acc, qtile=qtile):
            kblk = kb_ref[0, 0, pl.ds(kt * TQ, TQ), :]
            sc = jax.lax.dot_general(qtile, kblk, (((1,), (1,)), ((), ())),
                                     preferred_element_type=jnp.float32)
            pexp = jnp.exp(sc)
            vblk = vb_s[pl.ds(kt * TQ, TQ), :]
            return acc + jax.lax.dot_general(
                pexp.astype(jnp.bfloat16), vblk, (((1,), (0,)), ((), ())),
                preferred_element_type=jnp.float32)

        acc0 = jnp.zeros((TQ, 2 * DH), jnp.float32)
        acc = jax.lax.fori_loop(0, qt, body, acc0)
        kblk = kb_ref[0, 0, pl.ds(qt * TQ, TQ), :]
        sc = jax.lax.dot_general(qtile, kblk, (((1,), (1,)), ((), ())),
                                 preferred_element_type=jnp.float32)
        rowl = jax.lax.broadcasted_iota(jnp.int32, (TQ, TQ), 0)
        coll = jax.lax.broadcasted_iota(jnp.int32, (TQ, TQ), 1)
        pexp = jnp.exp(jnp.where(coll > rowl, NEG, sc))
        vblk = vb_s[pl.ds(qt * TQ, TQ), :]
        acc = acc + jax.lax.dot_general(
            pexp.astype(jnp.bfloat16), vblk, (((1,), (0,)), ((), ())),
            preferred_element_type=jnp.float32)
        local = acc[:, :DH] * (1.0 / acc[:, DH:DH + 1])

        gt = g[qt * TQ:(qt + 1) * TQ, :]
        vtile = vf_ref[0, 0, pl.ds(qt * TQ, TQ), :]
        yh = gt[:, 1:2] * local + gt[:, 2:3] * vtile
        part_ref[0, 0, pl.ds(qt * TQ, TQ), :] = jnp.concatenate(
            [yh, gt[:, 0:1], jnp.zeros((TQ, DH - 1), jnp.float32)],
            axis=1).astype(jnp.bfloat16)


BM = 1024


def _out_kernel(part_ref, rd_ref, wobf_ref, o_ref):
    h = pl.program_id(2)
    p = part_ref[0, 0]
    yh = p[:, :DH].astype(jnp.float32) + p[:, DH:DH + 1].astype(jnp.float32) * rd_ref[0]
    contrib = jax.lax.dot_general(yh.astype(jnp.bfloat16), wobf_ref[...],
                                  (((1,), (0,)), ((), ())),
                                  preferred_element_type=jnp.float32)

    @pl.when(h == 0)
    def _init():
        o_ref[0] = contrib

    @pl.when(h != 0)
    def _acc():
        o_ref[0] = o_ref[0] + contrib


@jax.jit
def kernel(x, Wqkv, Wout, slot_k_init, slot_v_init, Wg, bg, Wf, bf,
           log_tau_read, log_tau_write):
    ltau = log_tau_read.reshape(1, 1)
    bf2 = bf.reshape(1, 3)
    xbf = x.astype(jnp.bfloat16)
    Wqkvbf = Wqkv.astype(jnp.bfloat16)
    Wfbf = Wf.astype(jnp.bfloat16)
    WoTbf = Wout.T.astype(jnp.bfloat16)
    svflat = slot_v_init.reshape(H, K * DH)

    qsb, kb, vf, lg = pl.pallas_call(
        _proj_kernel,
        grid=(B, H // 2),
        in_specs=[
            pl.BlockSpec((1, 1), lambda b, h: (0, 0)),
            pl.BlockSpec((1, T, D), lambda b, h: (b, 0, 0)),
            pl.BlockSpec((1, T, D), lambda b, h: (b, 0, 0)),
            pl.BlockSpec((2 * DH, D), lambda b, h: (h, 0)),
            pl.BlockSpec((2 * DH, D), lambda b, h: (H // 2 + h, 0)),
            pl.BlockSpec((2 * DH, D), lambda b, h: (H + h, 0)),
            pl.BlockSpec((2, K, DH), lambda b, h: (h, 0, 0)),
        ],
        out_specs=[
            pl.BlockSpec((1, 2, T, DH), lambda b, h: (b, h, 0, 0)),
            pl.BlockSpec((1, 2, T, DH), lambda b, h: (b, h, 0, 0)),
            pl.BlockSpec((1, 2, T, DH), lambda b, h: (b, h, 0, 0)),
            pl.BlockSpec((2, T, K), lambda b, h: (b * (H // 2) + h, 0, 0)),
        ],
        out_shape=[
            jax.ShapeDtypeStruct((B, H, T, DH), jnp.bfloat16),
            jax.ShapeDtypeStruct((B, H, T, DH), jnp.bfloat16),
            jax.ShapeDtypeStruct((B, H, T, DH), jnp.float32),
            jax.ShapeDtypeStruct((BH, T, K), jnp.float32),
        ],
    )(ltau, x, xbf, Wqkv, Wqkvbf, Wqkvbf, slot_k_init)

    read = _sc_read(lg, svflat)

    part = pl.pallas_call(
        _attn_kernel,
        grid=(B, H),
        in_specs=[
            pl.BlockSpec((1, 1, T, DH), lambda b, h: (b, h, 0, 0)),
            pl.BlockSpec((1, 1, T, DH), lambda b, h: (b, h, 0, 0)),
            pl.BlockSpec((1, 1, T, DH), lambda b, h: (b, h, 0, 0)),
            pl.BlockSpec((3, DH), lambda b, h: (0, 0)),
            pl.BlockSpec((1, 3), lambda b, h: (0, 0)),
        ],
        out_specs=pl.BlockSpec((1, 1, T, 2 * DH), lambda b, h: (b, h, 0, 0)),
        out_shape=jax.ShapeDtypeStruct((B, H, T, 2 * DH), jnp.bfloat16),
        scratch_shapes=[pltpu.VMEM((T, 2 * DH), jnp.bfloat16)],
    )(qsb, kb, vf, Wfbf, bf2)

    y = pl.pallas_call(
        _out_kernel,
        grid=(B, T // BM, H),
        in_specs=[
            pl.BlockSpec((1, 1, BM, 2 * DH), lambda b, i, h: (b, h, i, 0)),
            pl.BlockSpec((1, BM, DH), lambda b, i, h: (b * H + h, i, 0)),
            pl.BlockSpec((DH, D), lambda b, i, h: (h, 0)),
        ],
        out_specs=pl.BlockSpec((1, BM, D), lambda b, i, h: (b, i, 0)),
        out_shape=jax.ShapeDtypeStruct((B, T, D), jnp.float32),
    )(part, read, WoTbf)
    return y


# speedup vs baseline: 1.0087x; 1.0087x over previous
import functools

import jax
import jax.numpy as jnp
import numpy as np
from jax.experimental import pallas as pl
from jax.experimental.pallas import tpu as pltpu
from jax.experimental.pallas import tpu_sc as plsc

B, T, D = 2, 2048, 1024
H, DH, K = 16, 64, 128
BH = B * H
RT = 8
TQ = 512
NEG = -1e30
L = 16
TCH = 256
SCALE = 1.0 / np.sqrt(DH)


def _proj_kernel(ltau_ref, x_ref, xbf_ref, wq_ref, wkbf_ref, wvbf_ref, sk_ref,
                 qsb_ref, kb_ref, vf_ref, lg_ref):
    xb = x_ref[0]
    xbf = xbf_ref[0]
    q = jax.lax.dot_general(xb, wq_ref[...], (((1,), (1,)), ((), ())),
                            preferred_element_type=jnp.float32)
    qsb_ref[0, 0] = (q * SCALE).astype(jnp.bfloat16)
    kb_ref[0, 0] = jax.lax.dot_general(
        xbf, wkbf_ref[...], (((1,), (1,)), ((), ())),
        preferred_element_type=jnp.float32).astype(jnp.bfloat16)
    vf_ref[0, 0] = jax.lax.dot_general(
        xbf, wvbf_ref[...], (((1,), (1,)), ((), ())),
        preferred_element_type=jnp.float32).astype(jnp.bfloat16)
    tau = jnp.exp(ltau_ref[0, 0])
    qn = q / jnp.maximum(jnp.sqrt(jnp.sum(q * q, axis=-1, keepdims=True)), 1e-12)
    sk = sk_ref[0]
    sk = sk / jnp.maximum(jnp.sqrt(jnp.sum(sk * sk, axis=-1, keepdims=True)), 1e-12)
    lg_ref[0] = jax.lax.dot_general(qn, sk, (((1,), (1,)), ((), ())),
                                    preferred_element_type=jnp.float32) * tau


def _bcast_lane(x, j):
    idx = jax.lax.broadcasted_iota(jnp.int32, (L,), 0) * 0 + j
    return jax.lax.gather(
        x, idx[:, None],
        jax.lax.GatherDimensionNumbers(offset_dims=(), collapsed_slice_dims=(0,),
                                       start_index_map=(0,)),
        (1,), mode=jax.lax.GatherScatterMode.PROMISE_IN_BOUNDS)


def _sc_read_body(lg_hbm, sv_hbm, out_hbm, lg_s, sv_s, ro_s, ib_s, eb_s, sem):
    c = jax.lax.axis_index("c")
    s = jax.lax.axis_index("s")
    wid = s * 2 + c
    head = jax.lax.rem(wid, H)
    pltpu.sync_copy(sv_hbm.at[head], sv_s)

    nvec = K // L

    def token_body(t, carry):
        lvec = [lg_s[t, pl.ds(i * L, L)] for i in range(nvec)]
        work = list(lvec)
        t8 = jnp.float32(0)
        for it in range(RT):
            m = work[0]
            for i in range(1, nvec):
                m = jnp.maximum(m, work[i])
            t8 = jnp.max(m)
            if it < RT - 1:
                mb = jnp.full((L,), t8, jnp.float32)
                work = [jnp.where(w >= mb, NEG, w) for w in work]
        t8b = jnp.full((L,), t8, jnp.float32)
        evec = [jnp.where(lv >= t8b, jnp.exp(lv - t8b), 0.0) for lv in lvec]
        den = evec[0]
        for i in range(1, nvec):
            den = den + evec[i]
        denb = jnp.full((L,), jnp.sum(den), jnp.float32)
        base = jnp.zeros((L,), jnp.int32)
        for i in range(nvec):
            msk = lvec[i] >= t8b
            mi = jnp.where(msk, 1, 0).astype(jnp.int32)
            rank = base + plsc.cumsum(mi) - 1
            iv = jax.lax.broadcasted_iota(jnp.int32, (L,), 0) + i * L
            plsc.store_scatter(ib_s, [rank], iv, mask=msk)
            plsc.store_scatter(eb_s, [rank], evec[i], mask=msk)
            base = base + plsc.all_reduce_population_count(msk)
        sel_i = ib_s[pl.ds(0, L)]
        sel_e = eb_s[pl.ds(0, L)]
        acc = [jnp.zeros((L,), jnp.float32) for _ in range(DH // L)]
        for j in range(RT):
            rowbase = _bcast_lane(sel_i, j) * DH
            ej = _bcast_lane(sel_e, j)
            for dc in range(DH // L):
                addr = rowbase + (dc * L + jax.lax.broadcasted_iota(jnp.int32, (L,), 0))
                acc[dc] = acc[dc] + ej * plsc.load_gather(sv_s, [addr])
        for dc in range(DH // L):
            ro_s[t, pl.ds(dc * L, L)] = acc[dc] / denb
        return carry

    for chunk in range(T // TCH):
        pltpu.sync_copy(lg_hbm.at[wid, pl.ds(chunk * TCH, TCH)], lg_s)
        jax.lax.fori_loop(0, TCH, token_body, jnp.int32(0))
        pltpu.sync_copy(ro_s, out_hbm.at[wid, pl.ds(chunk * TCH, TCH)])


_sc_read = functools.partial(
    pl.kernel,
    mesh=plsc.VectorSubcoreMesh(core_axis_name="c", subcore_axis_name="s"),
    compiler_params=pltpu.CompilerParams(needs_layout_passes=False),
    out_type=jax.ShapeDtypeStruct((BH, T, DH), jnp.float32),
    scratch_types=[
        pltpu.VMEM((TCH, K), jnp.float32),
        pltpu.VMEM((K * DH,), jnp.float32),
        pltpu.VMEM((TCH, DH), jnp.float32),
        pltpu.VMEM((160,), jnp.int32),
        pltpu.VMEM((160,), jnp.float32),
        pltpu.SemaphoreType.DMA,
    ],
)(_sc_read_body)


def _attn_kernel(qsb_ref, kb_ref, vf_ref, wfbf_ref, bf_ref, part_ref, vb_s):
    vb_s[:, :DH] = vf_ref[0, 0].astype(jnp.bfloat16)
    lane = jax.lax.broadcasted_iota(jnp.int32, (T, DH), 1)
    vb_s[:, DH:] = jnp.where(lane == 0, 1.0, 0.0).astype(jnp.bfloat16)

    qsb = qsb_ref[0, 0]
    gl = jax.lax.dot_general(qsb, wfbf_ref[...], (((1,), (1,)), ((), ())),
                             preferred_element_type=jnp.float32) / SCALE + bf_ref[...]
    gmx = jnp.max(gl, axis=-1, keepdims=True)
    ge = jnp.exp(gl - gmx)
    g = ge / jnp.sum(ge, axis=-1, keepdims=True)

    for qt in range(T // TQ):
        qtile = qsb_ref[0, 0, pl.ds(qt * TQ, TQ), :]

        def body(kt, acc, qtile=qtile):
            kblk = kb_ref[0, 0, pl.ds(kt * TQ, TQ), :]
            sc = jax.lax.dot_general(qtile, kblk, (((1,), (1,)), ((), ())),
                                     preferred_element_type=jnp.float32)
            pexp = jnp.exp(sc)
            vblk = vb_s[pl.ds(kt * TQ, TQ), :]
            return acc + jax.lax.dot_general(
                pexp.astype(jnp.bfloat16), vblk, (((1,), (0,)), ((), ())),
                preferred_element_type=jnp.float32)

        acc0 = jnp.zeros((TQ, 2 * DH), jnp.float32)
        acc = jax.lax.fori_loop(0, qt, body, acc0)
        kblk = kb_ref[0, 0, pl.ds(qt * TQ, TQ), :]
        sc = jax.lax.dot_general(qtile, kblk, (((1,), (1,)), ((), ())),
                                 preferred_element_type=jnp.float32)
        rowl = jax.lax.broadcasted_iota(jnp.int32, (TQ, TQ), 0)
        coll = jax.lax.broadcasted_iota(jnp.int32, (TQ, TQ), 1)
        pexp = jnp.exp(jnp.where(coll > rowl, NEG, sc))
        vblk = vb_s[pl.ds(qt * TQ, TQ), :]
        acc = acc + jax.lax.dot_general(
            pexp.astype(jnp.bfloat16), vblk, (((1,), (0,)), ((), ())),
            preferred_element_type=jnp.float32)
        local = acc[:, :DH] * (1.0 / acc[:, DH:DH + 1])

        gt = g[qt * TQ:(qt + 1) * TQ, :]
        vtile = vf_ref[0, 0, pl.ds(qt * TQ, TQ), :]
        yh = gt[:, 1:2] * local + gt[:, 2:3] * vtile
        part_ref[0, 0, pl.ds(qt * TQ, TQ), :] = jnp.concatenate(
            [yh, gt[:, 0:1], jnp.zeros((TQ, DH - 1), jnp.float32)],
            axis=1).astype(jnp.bfloat16)


BM = 1024


def _out_kernel(part_ref, rd_ref, wobf_ref, o_ref):
    h = pl.program_id(2)
    p = part_ref[0, 0]
    yh = p[:, :DH].astype(jnp.float32) + p[:, DH:DH + 1].astype(jnp.float32) * rd_ref[0]
    contrib = jax.lax.dot_general(yh.astype(jnp.bfloat16), wobf_ref[...],
                                  (((1,), (0,)), ((), ())),
                                  preferred_element_type=jnp.float32)

    @pl.when(h == 0)
    def _init():
        o_ref[0] = contrib

    @pl.when(h != 0)
    def _acc():
        o_ref[0] = o_ref[0] + contrib


@jax.jit
def kernel(x, Wqkv, Wout, slot_k_init, slot_v_init, Wg, bg, Wf, bf,
           log_tau_read, log_tau_write):
    ltau = log_tau_read.reshape(1, 1)
    bf2 = bf.reshape(1, 3)
    xbf = x.astype(jnp.bfloat16)
    Wqkvbf = Wqkv.astype(jnp.bfloat16)
    Wfbf = Wf.astype(jnp.bfloat16)
    WoTbf = Wout.T.astype(jnp.bfloat16)
    svflat = slot_v_init.reshape(H, K * DH)

    qsb, kb, vf, lg = pl.pallas_call(
        _proj_kernel,
        grid=(B, H),
        in_specs=[
            pl.BlockSpec((1, 1), lambda b, h: (0, 0)),
            pl.BlockSpec((1, T, D), lambda b, h: (b, 0, 0)),
            pl.BlockSpec((1, T, D), lambda b, h: (b, 0, 0)),
            pl.BlockSpec((DH, D), lambda b, h: (h, 0)),
            pl.BlockSpec((DH, D), lambda b, h: (H + h, 0)),
            pl.BlockSpec((DH, D), lambda b, h: (2 * H + h, 0)),
            pl.BlockSpec((1, K, DH), lambda b, h: (h, 0, 0)),
        ],
        out_specs=[
            pl.BlockSpec((1, 1, T, DH), lambda b, h: (b, h, 0, 0)),
            pl.BlockSpec((1, 1, T, DH), lambda b, h: (b, h, 0, 0)),
            pl.BlockSpec((1, 1, T, DH), lambda b, h: (b, h, 0, 0)),
            pl.BlockSpec((1, T, K), lambda b, h: (b * H + h, 0, 0)),
        ],
        out_shape=[
            jax.ShapeDtypeStruct((B, H, T, DH), jnp.bfloat16),
            jax.ShapeDtypeStruct((B, H, T, DH), jnp.bfloat16),
            jax.ShapeDtypeStruct((B, H, T, DH), jnp.bfloat16),
            jax.ShapeDtypeStruct((BH, T, K), jnp.float32),
        ],
    )(ltau, x, xbf, Wqkv, Wqkvbf, Wqkvbf, slot_k_init)

    read = _sc_read(lg, svflat)

    part = pl.pallas_call(
        _attn_kernel,
        grid=(B, H),
        in_specs=[
            pl.BlockSpec((1, 1, T, DH), lambda b, h: (b, h, 0, 0)),
            pl.BlockSpec((1, 1, T, DH), lambda b, h: (b, h, 0, 0)),
            pl.BlockSpec((1, 1, T, DH), lambda b, h: (b, h, 0, 0)),
            pl.BlockSpec((3, DH), lambda b, h: (0, 0)),
            pl.BlockSpec((1, 3), lambda b, h: (0, 0)),
        ],
        out_specs=pl.BlockSpec((1, 1, T, 2 * DH), lambda b, h: (b, h, 0, 0)),
        out_shape=jax.ShapeDtypeStruct((B, H, T, 2 * DH), jnp.bfloat16),
        scratch_shapes=[pltpu.VMEM((T, 2 * DH), jnp.bfloat16)],
    )(qsb, kb, vf, Wfbf, bf2)

    y = pl.pallas_call(
        _out_kernel,
        grid=(B, T // BM, H),
        in_specs=[
            pl.BlockSpec((1, 1, BM, 2 * DH), lambda b, i, h: (b, h, i, 0)),
            pl.BlockSpec((1, BM, DH), lambda b, i, h: (b * H + h, i, 0)),
            pl.BlockSpec((DH, D), lambda b, i, h: (h, 0)),
        ],
        out_specs=pl.BlockSpec((1, BM, D), lambda b, i, h: (b, i, 0)),
        out_shape=jax.ShapeDtypeStruct((B, T, D), jnp.float32),
    )(part, read, WoTbf)
    return y
